# baseline (speedup 1.0000x reference)
import functools

import jax
import jax.numpy as jnp
import numpy as np
from jax import lax
from jax.experimental import pallas as pl
from jax.experimental.pallas import tpu as pltpu
from jax.experimental.pallas import tpu_sc as plsc

_N = 8192
_ROWS = 2048
_NWORK = 32
_SC_ROWS = 256
_TC_ROWS = _ROWS - _SC_ROWS
_RPW = _SC_ROWS // _NWORK
_TC_BLOCK = 128
_K = 4095
_L = 16
_NB = 256
_BIG = np.int32(0x7FFFFFFF)


def _iota16():
    return lax.iota(jnp.int32, _L)


def _splat(s):
    return jnp.broadcast_to(s, (_L,))


def _scan_hist(hist, k_vec):
    zero = jnp.zeros((_L,), jnp.int32)
    run, bc, below, above = zero, zero, zero, _splat(_BIG)
    for c in range(_NB // 16):
        tot = hist[pl.ds(c * 16, 16)]
        for l in range(1, _L):
            tot = tot + hist[pl.ds(l * _NB + c * 16, 16)]
        cum = plsc.cumsum(tot) + run
        le = cum <= k_vec
        bc = bc + plsc.all_reduce_population_count(le)
        below = jnp.maximum(below, jnp.where(le, cum, 0))
        above = jnp.minimum(above, jnp.where(le, _BIG, cum))
        run = _splat(jnp.max(cum))
    return bc, _splat(jnp.max(below)), _splat(jnp.min(above))


def _sc_body(x_hbm, out_hbm, buf0, keys0, keys1, hist, outb):
    wid = lax.axis_index("s") * 2 + lax.axis_index("c")
    base = wid * _RPW
    iota = _iota16()
    lane_base = iota * _NB
    ones = jnp.ones((_L,), jnp.int32)
    zeros = jnp.zeros((_L,), jnp.int32)

    def zbody(i, _):
        hist[pl.ds(i * 16, 16)] = zeros
        return 0

    lax.fori_loop(0, (_NB * _L) // 16, zbody, 0)

    def process(buf, src, dst, j):
        def l0hist(i, _):
            for u in range(4):
                off = (i * 4 + u) * 16
                bits = buf[pl.ds(off, 16)]
                key = bits ^ (lax.shift_right_arithmetic(bits, 31) & _BIG)
                bucket = lax.shift_right_arithmetic(key, 24) + 128
                plsc.addupdate_scatter(hist, [lane_base + bucket], ones)
            return 0

        lax.fori_loop(0, _N // 64, l0hist, 0)
        k_vec = _splat(jnp.int32(_K))
        b0, below, above = _scan_hist(hist, k_vec)
        k_vec = k_vec - below
        n_vec = above - below
        prefix = lax.shift_left(b0 - 128, 24)

        def zclr(i, _):
            for u in range(4):
                hist[pl.ds((i * 4 + u) * 16, 16)] = zeros
            return 0

        lax.fori_loop(0, (_NB * _L) // 64, zclr, 0)

        def l0comp(i, wb):
            for u in range(4):
                off = (i * 4 + u) * 16
                bits = buf[pl.ds(off, 16)]
                key = bits ^ (lax.shift_right_arithmetic(bits, 31) & _BIG)
                bucket = lax.shift_right_arithmetic(key, 24) + 128
                m = bucket == b0
                mi = m.astype(jnp.int32)
                pos = jnp.minimum(plsc.cumsum(mi) - mi + wb, _N - 1)
                plsc.store_scatter(dst, [pos], key, mask=m)
                wb = wb + plsc.all_reduce_population_count(m)
            return wb

        lax.fori_loop(0, _N // 64, l0comp, zeros)

        def tail_small(_):
            v = dst[pl.ds(0, 16)]
            v = jnp.where(iota < n_vec, v, _splat(_BIG))
            sv = jnp.sort(v)
            out_bits = sv ^ (lax.shift_right_arithmetic(sv, 31) & _BIG)
            plsc.store_scatter(
                outb, [_splat(j)], out_bits, mask=iota == k_vec
            )
            return 0

        def tail_full(_):
            kv, nv, pfx = k_vec, n_vec, prefix
            cur, other = dst, src
            for shift in (16, 8, 0):
                nv = jnp.minimum(nv, _N)
                trips = lax.shift_right_logical(jnp.max(nv) + 15, 4)

                def lhist(i, _, cur=cur, shift=shift, nv=nv):
                    key = cur[pl.ds(i * 16, 16)]
                    valid = (iota + i * 16) < nv
                    bucket = lax.shift_right_logical(key, shift) & 0xFF
                    plsc.addupdate_scatter(
                        hist, [lane_base + bucket], ones, mask=valid
                    )
                    return 0

                lax.fori_loop(0, trips, lhist, 0)
                b, below, above = _scan_hist(hist, kv)
                kv = kv - below
                pfx = pfx | lax.shift_left(b, shift)

                def lcomp(i, wb, cur=cur, other=other, shift=shift, nv=nv, b=b):
                    key = cur[pl.ds(i * 16, 16)]
                    valid = (iota + i * 16) < nv
                    bucket = lax.shift_right_logical(key, shift) & 0xFF
                    plsc.store_scatter(
                        hist, [lane_base + bucket], zeros, mask=valid
                    )
                    if shift != 0:
                        m = jnp.logical_and(bucket == b, valid)
                        mi = m.astype(jnp.int32)
                        pos = jnp.minimum(plsc.cumsum(mi) - mi + wb, _N - 1)
                        plsc.store_scatter(other, [pos], key, mask=m)
                        return wb + plsc.all_reduce_population_count(m)
                    return wb

                lax.fori_loop(0, trips, lcomp, zeros)
                nv = above - below
                cur, other = other, cur

            out_bits = pfx ^ (lax.shift_right_arithmetic(pfx, 31) & _BIG)
            plsc.store_scatter(outb, [_splat(j)], out_bits, mask=iota == 0)
            return 0

        lax.cond(jnp.max(n_vec) <= _L, tail_small, tail_full, 0)

    def rows(j, _):
        pltpu.sync_copy(x_hbm.at[base + j], buf0)
        process(buf0, keys0, keys1, j)
        return 0

    lax.fori_loop(0, _RPW, rows, 0)
    pltpu.sync_copy(outb, out_hbm.at[pl.ds(base, _RPW)])


def _tc_body(x_ref, o_ref):
    xb = x_ref[...]
    bits = jax.lax.bitcast_convert_type(xb, jnp.int32)
    keys = bits ^ (lax.shift_right_arithmetic(bits, 31) & jnp.int32(0x7FFFFFFF))
    lo = jnp.full((_TC_BLOCK, 1), jnp.int32(-(2**31)))
    k = jnp.full((_TC_BLOCK, 1), _K, jnp.int32)
    for b in range(31, -1, -1):
        mid = lo + (jnp.int32(1) << jnp.int32(b))
        c = jnp.sum(
            jnp.logical_and(keys >= lo, keys < mid).astype(jnp.int32),
            axis=1,
            keepdims=True,
        )
        go_hi = k >= c
        lo = jnp.where(go_hi, mid, lo)
        k = jnp.where(go_hi, k - c, k)
    key_sel = lo[:, 0]
    out_bits = key_sel ^ (
        lax.shift_right_arithmetic(key_sel, 31) & jnp.int32(0x7FFFFFFF)
    )
    o_ref[0, 0, :] = jax.lax.bitcast_convert_type(out_bits, jnp.float32)


def kernel(x):
    b, h, n = x.shape
    xf = x.reshape(b * h, n)
    xi = jax.lax.bitcast_convert_type(xf[:_SC_ROWS], jnp.int32)
    mesh = plsc.VectorSubcoreMesh(core_axis_name="c", subcore_axis_name="s")
    sc_f = functools.partial(
        pl.kernel,
        mesh=mesh,
        compiler_params=pltpu.CompilerParams(
            needs_layout_passes=False, use_tc_tiling_on_sc=False
        ),
        out_type=jax.ShapeDtypeStruct((_SC_ROWS,), jnp.int32),
        scratch_types=[
            pltpu.VMEM((_N,), jnp.int32),
            pltpu.VMEM((_N,), jnp.int32),
            pltpu.VMEM((_N,), jnp.int32),
            pltpu.VMEM((_NB * _L,), jnp.int32),
            pltpu.VMEM((_RPW,), jnp.int32),
        ],
    )(_sc_body)
    out_sc = jax.lax.bitcast_convert_type(sc_f(xi), jnp.float32)
    grid = _TC_ROWS // _TC_BLOCK
    out_tc = pl.pallas_call(
        _tc_body,
        grid=(grid,),
        in_specs=[pl.BlockSpec((_TC_BLOCK, n), lambda i: (i, 0))],
        out_specs=pl.BlockSpec((1, 1, _TC_BLOCK), lambda i: (i, 0, 0)),
        out_shape=jax.ShapeDtypeStruct((grid, 1, _TC_BLOCK), jnp.float32),
    )(xf[_SC_ROWS:]).reshape(_TC_ROWS)
    return jnp.concatenate([out_sc, out_tc]).reshape(b, h)


# speedup vs baseline: 1.0690x; 1.0690x over previous
import functools

import jax
import jax.numpy as jnp
import numpy as np
from jax import lax
from jax.experimental import pallas as pl
from jax.experimental.pallas import tpu as pltpu
from jax.experimental.pallas import tpu_sc as plsc

_N = 8192
_ROWS = 2048
_NWORK = 32
_SC_ROWS = 512
_TC_ROWS = _ROWS - _SC_ROWS
_RPW = _SC_ROWS // _NWORK
_TC_BLOCK = 128
_K = 4095
_L = 16
_NB = 256
_BIG = np.int32(0x7FFFFFFF)


def _iota16():
    return lax.iota(jnp.int32, _L)


def _splat(s):
    return jnp.broadcast_to(s, (_L,))


def _scan_hist(hist, k_vec):
    zero = jnp.zeros((_L,), jnp.int32)
    run, bc, below, above = zero, zero, zero, _splat(_BIG)
    for c in range(_NB // 16):
        tot = hist[pl.ds(c * 16, 16)]
        for l in range(1, _L):
            tot = tot + hist[pl.ds(l * _NB + c * 16, 16)]
        cum = plsc.cumsum(tot) + run
        le = cum <= k_vec
        bc = bc + plsc.all_reduce_population_count(le)
        below = jnp.maximum(below, jnp.where(le, cum, 0))
        above = jnp.minimum(above, jnp.where(le, _BIG, cum))
        run = _splat(jnp.max(cum))
    return bc, _splat(jnp.max(below)), _splat(jnp.min(above))


def _sc_body(x_hbm, out_hbm, buf0, keys0, keys1, hist, outb):
    wid = lax.axis_index("s") * 2 + lax.axis_index("c")
    base = wid * _RPW
    iota = _iota16()
    lane_base = iota * _NB
    ones = jnp.ones((_L,), jnp.int32)
    zeros = jnp.zeros((_L,), jnp.int32)

    def zbody(i, _):
        hist[pl.ds(i * 16, 16)] = zeros
        return 0

    lax.fori_loop(0, (_NB * _L) // 16, zbody, 0)

    def process(buf, src, dst, j):
        def l0hist(i, _):
            for u in range(4):
                off = (i * 4 + u) * 16
                bits = buf[pl.ds(off, 16)]
                key = bits ^ (lax.shift_right_arithmetic(bits, 31) & _BIG)
                bucket = lax.shift_right_arithmetic(key, 24) + 128
                plsc.addupdate_scatter(hist, [lane_base + bucket], ones)
            return 0

        lax.fori_loop(0, _N // 64, l0hist, 0)
        k_vec = _splat(jnp.int32(_K))
        b0, below, above = _scan_hist(hist, k_vec)
        k_vec = k_vec - below
        n_vec = above - below
        prefix = lax.shift_left(b0 - 128, 24)

        def zclr(i, _):
            for u in range(4):
                hist[pl.ds((i * 4 + u) * 16, 16)] = zeros
            return 0

        lax.fori_loop(0, (_NB * _L) // 64, zclr, 0)

        def l0comp(i, wb):
            for u in range(4):
                off = (i * 4 + u) * 16
                bits = buf[pl.ds(off, 16)]
                key = bits ^ (lax.shift_right_arithmetic(bits, 31) & _BIG)
                bucket = lax.shift_right_arithmetic(key, 24) + 128
                m = bucket == b0
                mi = m.astype(jnp.int32)
                pos = jnp.minimum(plsc.cumsum(mi) - mi + wb, _N - 1)
                plsc.store_scatter(dst, [pos], key, mask=m)
                wb = wb + plsc.all_reduce_population_count(m)
            return wb

        lax.fori_loop(0, _N // 64, l0comp, zeros)

        def tail_small(_):
            v = dst[pl.ds(0, 16)]
            v = jnp.where(iota < n_vec, v, _splat(_BIG))
            sv = jnp.sort(v)
            out_bits = sv ^ (lax.shift_right_arithmetic(sv, 31) & _BIG)
            plsc.store_scatter(
                outb, [_splat(j)], out_bits, mask=iota == k_vec
            )
            return 0

        def tail_full(_):
            kv, nv, pfx = k_vec, n_vec, prefix
            cur, other = dst, src
            for shift in (16, 8, 0):
                nv = jnp.minimum(nv, _N)
                trips = lax.shift_right_logical(jnp.max(nv) + 15, 4)

                def lhist(i, _, cur=cur, shift=shift, nv=nv):
                    key = cur[pl.ds(i * 16, 16)]
                    valid = (iota + i * 16) < nv
                    bucket = lax.shift_right_logical(key, shift) & 0xFF
                    plsc.addupdate_scatter(
                        hist, [lane_base + bucket], ones, mask=valid
                    )
                    return 0

                lax.fori_loop(0, trips, lhist, 0)
                b, below, above = _scan_hist(hist, kv)
                kv = kv - below
                pfx = pfx | lax.shift_left(b, shift)

                def lcomp(i, wb, cur=cur, other=other, shift=shift, nv=nv, b=b):
                    key = cur[pl.ds(i * 16, 16)]
                    valid = (iota + i * 16) < nv
                    bucket = lax.shift_right_logical(key, shift) & 0xFF
                    plsc.store_scatter(
                        hist, [lane_base + bucket], zeros, mask=valid
                    )
                    if shift != 0:
                        m = jnp.logical_and(bucket == b, valid)
                        mi = m.astype(jnp.int32)
                        pos = jnp.minimum(plsc.cumsum(mi) - mi + wb, _N - 1)
                        plsc.store_scatter(other, [pos], key, mask=m)
                        return wb + plsc.all_reduce_population_count(m)
                    return wb

                lax.fori_loop(0, trips, lcomp, zeros)
                nv = above - below
                cur, other = other, cur

            out_bits = pfx ^ (lax.shift_right_arithmetic(pfx, 31) & _BIG)
            plsc.store_scatter(outb, [_splat(j)], out_bits, mask=iota == 0)
            return 0

        lax.cond(jnp.max(n_vec) <= _L, tail_small, tail_full, 0)

    def rows(j, _):
        pltpu.sync_copy(x_hbm.at[base + j], buf0)
        process(buf0, keys0, keys1, j)
        return 0

    lax.fori_loop(0, _RPW, rows, 0)
    pltpu.sync_copy(outb, out_hbm.at[pl.ds(base, _RPW)])


def _tc_body(x_ref, o_ref):
    xb = x_ref[...]
    bits = jax.lax.bitcast_convert_type(xb, jnp.int32)
    keys = bits ^ (lax.shift_right_arithmetic(bits, 31) & jnp.int32(0x7FFFFFFF))
    lo = jnp.full((_TC_BLOCK, 1), jnp.int32(-(2**31)))
    k = jnp.full((_TC_BLOCK, 1), _K, jnp.int32)
    for b in range(31, -1, -1):
        mid = lo + (jnp.int32(1) << jnp.int32(b))
        c = jnp.sum(
            jnp.logical_and(keys >= lo, keys < mid).astype(jnp.int32),
            axis=1,
            keepdims=True,
        )
        go_hi = k >= c
        lo = jnp.where(go_hi, mid, lo)
        k = jnp.where(go_hi, k - c, k)
    key_sel = lo[:, 0]
    out_bits = key_sel ^ (
        lax.shift_right_arithmetic(key_sel, 31) & jnp.int32(0x7FFFFFFF)
    )
    o_ref[0, 0, :] = jax.lax.bitcast_convert_type(out_bits, jnp.float32)


def kernel(x):
    b, h, n = x.shape
    xf = x.reshape(b * h, n)
    xi = jax.lax.bitcast_convert_type(xf[:_SC_ROWS], jnp.int32)
    mesh = plsc.VectorSubcoreMesh(core_axis_name="c", subcore_axis_name="s")
    sc_f = functools.partial(
        pl.kernel,
        mesh=mesh,
        compiler_params=pltpu.CompilerParams(
            needs_layout_passes=False, use_tc_tiling_on_sc=False
        ),
        out_type=jax.ShapeDtypeStruct((_SC_ROWS,), jnp.int32),
        scratch_types=[
            pltpu.VMEM((_N,), jnp.int32),
            pltpu.VMEM((_N,), jnp.int32),
            pltpu.VMEM((_N,), jnp.int32),
            pltpu.VMEM((_NB * _L,), jnp.int32),
            pltpu.VMEM((_RPW,), jnp.int32),
        ],
    )(_sc_body)
    out_sc = jax.lax.bitcast_convert_type(sc_f(xi), jnp.float32)
    grid = _TC_ROWS // _TC_BLOCK
    out_tc = pl.pallas_call(
        _tc_body,
        grid=(grid,),
        in_specs=[pl.BlockSpec((_TC_BLOCK, n), lambda i: (i, 0))],
        out_specs=pl.BlockSpec((1, 1, _TC_BLOCK), lambda i: (i, 0, 0)),
        out_shape=jax.ShapeDtypeStruct((grid, 1, _TC_BLOCK), jnp.float32),
    )(xf[_SC_ROWS:]).reshape(_TC_ROWS)
    return jnp.concatenate([out_sc, out_tc]).reshape(b, h)
